# baseline (speedup 1.0000x reference)
import functools

import jax
import jax.numpy as jnp
from jax import lax
from jax.experimental import pallas as pl
from jax.experimental.pallas import tpu as pltpu
from jax.experimental.pallas import tpu_sc as plsc

VOCAB_N = 1000000
EMB_D = 64
OUT_D = 16
SEQ = 200
BATCH_N = 4096

NUM_CORES = 2
NUM_SUBCORES = 16
NW = NUM_CORES * NUM_SUBCORES
COLS_PER_W = BATCH_N // NW
CHUNK_COLS = 8
N_CHUNKS = COLS_PER_W // CHUNK_COLS
SEQ_PAD = 208
IDX_ROWS = CHUNK_COLS * SEQ_PAD // 128
ROWS_PER_CHUNK = IDX_ROWS * 128

PROJ_BLK = 8000


def _proj_body(emb_ref, wt_ref, out_ref):
    out_ref[...] = jnp.dot(
        emb_ref[...], wt_ref[...], preferred_element_type=jnp.float32
    )


@jax.jit
def _project(emb, wt):
    return pl.pallas_call(
        _proj_body,
        grid=(VOCAB_N // PROJ_BLK,),
        in_specs=[
            pl.BlockSpec((PROJ_BLK, EMB_D), lambda i: (i, 0)),
            pl.BlockSpec((EMB_D, OUT_D), lambda i: (0, 0)),
        ],
        out_specs=pl.BlockSpec((PROJ_BLK, OUT_D), lambda i: (i, 0)),
        out_shape=jax.ShapeDtypeStruct((VOCAB_N, OUT_D), jnp.float32),
    )(emb, wt)


def _sc_body(text_hbm, tab_hbm, b_hbm, out_hbm, idx_v, rows_v, st_v, b_v, sem):
    wid = lax.axis_index("s") * NUM_CORES + lax.axis_index("c")
    pltpu.sync_copy(b_hbm, b_v)
    bias = b_v[...]

    def chunk_body(g, carry):
        pltpu.sync_copy(text_hbm.at[wid * N_CHUNKS + g], idx_v)
        copies = [
            pltpu.async_copy(
                tab_hbm.at[idx_v.at[j]],
                rows_v.at[pl.ds(j * 128, 128)],
                sem,
            )
            for j in range(IDX_ROWS)
        ]
        for cp in copies:
            cp.wait()

        for c in range(CHUNK_COLS):
            base = c * SEQ_PAD

            def row_body(i, accs, base=base):
                a0, a1, a2, a3 = accs
                r = base + i * 4
                return (
                    a0 + rows_v[r],
                    a1 + rows_v[r + 1],
                    a2 + rows_v[r + 2],
                    a3 + rows_v[r + 3],
                )

            z = jnp.zeros((OUT_D,), jnp.float32)
            a0, a1, a2, a3 = lax.fori_loop(
                0, SEQ_PAD // 4, row_body, (z, z, z, z)
            )
            st_v[c] = (a0 + a1) + (a2 + a3) + bias

        pltpu.sync_copy(
            st_v,
            out_hbm.at[pl.ds(wid * COLS_PER_W + g * CHUNK_COLS, CHUNK_COLS)],
        )
        return carry

    lax.fori_loop(0, N_CHUNKS, chunk_body, 0)


@jax.jit
def _sc_pool(text3, table, b):
    mesh = plsc.VectorSubcoreMesh(
        core_axis_name="c", subcore_axis_name="s", num_cores=NUM_CORES
    )
    run = functools.partial(
        pl.kernel,
        out_type=jax.ShapeDtypeStruct((BATCH_N, OUT_D), jnp.float32),
        mesh=mesh,
        scratch_types=[
            pltpu.VMEM((IDX_ROWS, 128), jnp.int32),
            pltpu.VMEM((ROWS_PER_CHUNK, OUT_D), jnp.float32),
            pltpu.VMEM((CHUNK_COLS, OUT_D), jnp.float32),
            pltpu.VMEM((OUT_D,), jnp.float32),
            pltpu.SemaphoreType.DMA,
        ],
        compiler_params=pltpu.CompilerParams(use_tc_tiling_on_sc=False),
    )(_sc_body)
    return run(text3, table, b)


def kernel(text, emb, W, b):
    wt = (W.T * (1.0 / SEQ)).astype(jnp.float32)
    table = _project(emb, wt)
    textT = jnp.transpose(text.astype(jnp.int32))
    textp = jnp.pad(textT, ((0, 0), (0, SEQ_PAD - SEQ)))
    text3 = textp.reshape(BATCH_N // CHUNK_COLS, IDX_ROWS, 128)
    return _sc_pool(text3, table, b)
